# baseline (speedup 1.0000x reference)
import functools

import jax
import jax.numpy as jnp
from jax import lax
from jax.experimental import pallas as pl
from jax.experimental.pallas import tpu as pltpu
from jax.experimental.pallas import tpu_sc as plsc

EMB = 64
BATCH = 4096
SEQ = 200
B_TOTAL = BATCH * SEQ
NUM_WORKERS = 32
B_PER_W = B_TOTAL // NUM_WORKERS
CHUNK = 512
N_CHUNKS = B_PER_W // CHUNK

_mesh = plsc.VectorSubcoreMesh(core_axis_name="c", subcore_axis_name="s")


@functools.partial(
    pl.kernel,
    mesh=_mesh,
    out_type=jax.ShapeDtypeStruct((B_TOTAL, EMB), jnp.float32),
    scratch_types=[
        pltpu.VMEM((CHUNK,), jnp.int32),
        pltpu.VMEM((CHUNK, EMB), jnp.float32),
        pltpu.SemaphoreType.DMA,
    ],
    compiler_params=pltpu.CompilerParams(use_tc_tiling_on_sc=False),
)
def _gather_all(idx_hbm, table_hbm, out_hbm, idx_v, rows_v, sem):
    wid = lax.axis_index("s") * 2 + lax.axis_index("c")
    base = pl.multiple_of(wid * B_PER_W, CHUNK)

    def body(i, _):
        off = pl.multiple_of(base + i * CHUNK, CHUNK)
        pltpu.sync_copy(idx_hbm.at[pl.ds(off, CHUNK)], idx_v)
        pltpu.async_copy(table_hbm.at[idx_v], rows_v, sem).wait()
        pltpu.sync_copy(rows_v, out_hbm.at[pl.ds(off, CHUNK)])
        return ()

    lax.fori_loop(0, N_CHUNKS, body, ())


def kernel(x, table):
    flat = x.reshape(B_TOTAL)
    out = _gather_all(flat, table)
    return out.reshape(BATCH, SEQ, EMB)


# speedup vs baseline: 1.0441x; 1.0441x over previous
import functools

import jax
import jax.numpy as jnp
from jax import lax
from jax.experimental import pallas as pl
from jax.experimental.pallas import tpu as pltpu
from jax.experimental.pallas import tpu_sc as plsc

EMB = 64
BATCH = 4096
SEQ = 200
B_TOTAL = BATCH * SEQ
NUM_WORKERS = 32
B_PER_W = B_TOTAL // NUM_WORKERS
CHUNK = 256
N_CHUNKS = B_PER_W // CHUNK
NBUF = 4
NOUT = N_CHUNKS // NBUF

_mesh = plsc.VectorSubcoreMesh(core_axis_name="c", subcore_axis_name="s")

_scratch = (
    [pltpu.VMEM((CHUNK,), jnp.int32) for _ in range(NBUF)]
    + [pltpu.VMEM((CHUNK, EMB), jnp.float32) for _ in range(NBUF)]
    + [pltpu.SemaphoreType.DMA for _ in range(2 * NBUF)]
)


@functools.partial(
    pl.kernel,
    mesh=_mesh,
    out_type=jax.ShapeDtypeStruct((B_TOTAL, EMB), jnp.float32),
    scratch_types=_scratch,
    compiler_params=pltpu.CompilerParams(use_tc_tiling_on_sc=False),
)
def _gather_all(idx_hbm, table_hbm, out_hbm, *scr):
    idx_v = scr[0:NBUF]
    rows_v = scr[NBUF : 2 * NBUF]
    gsem = scr[2 * NBUF : 3 * NBUF]
    wsem = scr[3 * NBUF : 4 * NBUF]

    wid = lax.axis_index("s") * 2 + lax.axis_index("c")
    base = wid * B_PER_W

    def issue_gather(i, b):
        off = pl.multiple_of(base + i * CHUNK, CHUNK)
        pltpu.sync_copy(idx_hbm.at[pl.ds(off, CHUNK)], idx_v[b])
        pltpu.async_copy(table_hbm.at[idx_v[b]], rows_v[b], gsem[b])

    def wait_gather(b):
        pltpu.make_async_copy(table_hbm.at[idx_v[b]], rows_v[b], gsem[b]).wait()

    def issue_write(i, b):
        off = pl.multiple_of(base + i * CHUNK, CHUNK)
        pltpu.async_copy(rows_v[b], out_hbm.at[pl.ds(off, CHUNK)], wsem[b])

    def wait_write(b):
        pltpu.make_async_copy(
            rows_v[b], out_hbm.at[pl.ds(base, CHUNK)], wsem[b]
        ).wait()

    for b in range(NBUF):
        issue_gather(b, b)

    def outer(g, _):
        first = g * NBUF
        for b in range(NBUF):
            wait_gather(b)
            issue_write(first + b, b)
        for b in range(NBUF):
            wait_write(b)
            issue_gather(first + NBUF + b, b)
        return ()

    lax.fori_loop(0, NOUT - 1, outer, ())

    first = (NOUT - 1) * NBUF
    for b in range(NBUF):
        wait_gather(b)
        issue_write(first + b, b)
    for b in range(NBUF):
        wait_write(b)


def kernel(x, table):
    flat = x.reshape(B_TOTAL)
    out = _gather_all(flat, table)
    return out.reshape(BATCH, SEQ, EMB)
